# baseline (speedup 1.0000x reference)
import functools

import jax
import jax.numpy as jnp
from jax import lax
from jax.experimental import pallas as pl
from jax.experimental.pallas import tpu as pltpu
from jax.experimental.pallas import tpu_sc as plsc

_B = 16384
_S = 200
_D = 32
_NTOK = _B * _S
_NBLK = _NTOK // 128
_NW = 32
_BPW = _NBLK // _NW
_NPAIR = _BPW // 2

_mesh = plsc.VectorSubcoreMesh(core_axis_name="c", subcore_axis_name="s")


@functools.partial(
    pl.kernel,
    mesh=_mesh,
    compiler_params=pltpu.CompilerParams(use_tc_tiling_on_sc=False,
                                         needs_layout_passes=False),
    out_type=jax.ShapeDtypeStruct((_NTOK * _D,), jnp.float32),
    scratch_types=[
        pltpu.VMEM((2, 128), jnp.int32),
        pltpu.VMEM((2, 128), jnp.int32),
        pltpu.VMEM((128, _D), jnp.float32),
        pltpu.VMEM((128, _D), jnp.float32),
        pltpu.VMEM((4, 1024), jnp.float32),
        pltpu.VMEM((4, 1024), jnp.float32),
        pltpu.SemaphoreType.DMA,
        pltpu.SemaphoreType.DMA,
        pltpu.SemaphoreType.DMA,
        pltpu.SemaphoreType.DMA,
        pltpu.SemaphoreType.DMA,
        pltpu.SemaphoreType.DMA,
    ],
)
def _gather_kernel(idx_hbm, table_hbm, out_hbm,
                   ipx0, ipx1, rows0, rows1, tb0, tb1,
                   isem0, isem1, gsem0, gsem1, osem0, osem1):
    ipxs = (ipx0, ipx1)
    rows_vs = (rows0, rows1)
    tbs = (tb0, tb1)
    isems = (isem0, isem1)
    gsems = (gsem0, gsem1)
    osems = (osem0, osem1)

    wid = lax.axis_index("s") * 2 + lax.axis_index("c")
    base = wid * _BPW

    iota = lax.iota(jnp.int32, 16)
    rvs = [iota + 16 * h for h in range(8)]

    def start_idx(p, q):
        pltpu.async_copy(idx_hbm.at[pl.ds(base + 2 * p, 2)], ipxs[q],
                         isems[q])

    def wait_idx(q):
        pltpu.make_async_copy(idx_hbm.at[pl.ds(0, 2)], ipxs[q],
                              isems[q]).wait()

    def start_gather(q, jj, s):
        pltpu.async_copy(table_hbm.at[ipxs[q].at[jj]], rows_vs[s],
                         gsems[s])

    def wait_gather(q, jj, s):
        pltpu.make_async_copy(table_hbm.at[ipxs[q].at[jj]], rows_vs[s],
                              gsems[s]).wait()

    def transpose(s):
        for d in range(_D):
            cv = jnp.full((16,), d, jnp.int32)
            dt, di = d // 8, d % 8
            for h in range(8):
                v = plsc.load_gather(rows_vs[s], [rvs[h], cv])
                tbs[s][dt, pl.ds(di * 128 + 16 * h, 16)] = v

    def start_stores(r, s):
        s_pos = (r // 1024) * 8 + r % 8
        bt = (r // 8) % 128
        o = (s_pos * 4 * 128 + bt) * 1024
        for dt in range(4):
            pltpu.async_copy(tbs[s].at[dt],
                             out_hbm.at[pl.ds(o + dt * 131072, 1024)],
                             osems[s])

    def wait_stores(s):
        for dt in range(4):
            pltpu.make_async_copy(tbs[s].at[dt],
                                  out_hbm.at[pl.ds(dt * 1024, 1024)],
                                  osems[s]).wait()

    def emit_pair(p, q, prefetch=True):
        start_gather(q, 1, 1)
        wait_gather(q, 0, 0)
        wait_stores(0)
        transpose(0)
        start_stores(base + 2 * p, 0)
        wait_idx(1 - q)
        start_gather(1 - q, 0, 0)
        wait_gather(q, 1, 1)
        wait_stores(1)
        transpose(1)
        start_stores(base + 2 * p + 1, 1)
        if prefetch:
            start_idx(p + 2, q)

    start_idx(0, 0)
    wait_idx(0)
    start_gather(0, 0, 0)
    start_idx(1, 1)

    start_gather(0, 1, 1)
    wait_gather(0, 0, 0)
    transpose(0)
    start_stores(base + 0, 0)
    wait_idx(1)
    start_gather(1, 0, 0)
    wait_gather(0, 1, 1)
    transpose(1)
    start_stores(base + 1, 1)
    start_idx(2, 0)

    def body(t, carry):
        emit_pair(2 * t + 1, 1)
        emit_pair(2 * t + 2, 0)
        return carry

    lax.fori_loop(0, (_NPAIR - 4) // 2, body, 0)

    emit_pair(_NPAIR - 3, 1)
    emit_pair(_NPAIR - 2, 0, prefetch=False)
    p = _NPAIR - 1
    start_gather(1, 1, 1)
    wait_gather(1, 0, 0)
    wait_stores(0)
    transpose(0)
    start_stores(base + 2 * p, 0)
    wait_gather(1, 1, 1)
    wait_stores(1)
    transpose(1)
    start_stores(base + 2 * p + 1, 1)
    wait_stores(0)
    wait_stores(1)


def kernel(token_ids, embeddings):
    ids = jnp.transpose(
        token_ids.astype(jnp.int32).reshape(128, 128, 25, 8),
        (2, 0, 3, 1)).reshape(_NBLK, 128)
    out = _gather_kernel(ids, embeddings)
    t = out.reshape(_S, 4, 128, 8, 128)
    return jnp.transpose(t, (2, 4, 0, 1, 3)).reshape(_B, _S, _D)


# speedup vs baseline: 1.1596x; 1.1596x over previous
import functools

import jax
import jax.numpy as jnp
from jax import lax
from jax.experimental import pallas as pl
from jax.experimental.pallas import tpu as pltpu
from jax.experimental.pallas import tpu_sc as plsc

_B = 16384
_S = 200
_D = 32
_NTOK = _B * _S
_NBLK = _NTOK // 128
_NW = 32
_BPW = _NBLK // _NW
_NSLOT = 8
_NGRP = _BPW // _NSLOT

_mesh = plsc.VectorSubcoreMesh(core_axis_name="c", subcore_axis_name="s")


@functools.partial(
    pl.kernel,
    mesh=_mesh,
    compiler_params=pltpu.CompilerParams(use_tc_tiling_on_sc=False,
                                         needs_layout_passes=False),
    out_type=jax.ShapeDtypeStruct((_NTOK * _D,), jnp.float32),
    scratch_types=(
        [pltpu.VMEM((1, 128), jnp.int32) for _ in range(_NSLOT)]
        + [pltpu.VMEM((128, _D), jnp.float32) for _ in range(_NSLOT)]
        + [pltpu.VMEM((4096,), jnp.float32) for _ in range(_NSLOT)]
        + [pltpu.SemaphoreType.DMA for _ in range(3 * _NSLOT)]
    ),
)
def _gather_kernel(idx_hbm, table_hbm, out_hbm, *scratch):
    ixs = scratch[0:_NSLOT]
    rows_vs = scratch[_NSLOT:2 * _NSLOT]
    tbs = scratch[2 * _NSLOT:3 * _NSLOT]
    isems = scratch[3 * _NSLOT:4 * _NSLOT]
    gsems = scratch[4 * _NSLOT:5 * _NSLOT]
    osems = scratch[5 * _NSLOT:6 * _NSLOT]

    wid = lax.axis_index("s") * 2 + lax.axis_index("c")
    base = wid * _BPW

    iota = lax.iota(jnp.int32, 16)
    rvs = [iota + 16 * h for h in range(8)]

    def start_idx(r, j):
        pltpu.async_copy(idx_hbm.at[pl.ds(base + r, 1)], ixs[j], isems[j])

    def wait_idx(j):
        pltpu.make_async_copy(idx_hbm.at[pl.ds(0, 1)], ixs[j],
                              isems[j]).wait()

    def start_gather(j):
        pltpu.async_copy(table_hbm.at[ixs[j].at[0]], rows_vs[j], gsems[j])

    def wait_gather(j):
        pltpu.make_async_copy(table_hbm.at[ixs[j].at[0]], rows_vs[j],
                              gsems[j]).wait()

    def transpose(j):
        def tbody(d, c):
            cv = jnp.full((16,), 1, jnp.int32) * d
            o = (d // 8) * 1024 + (d % 8) * 128
            for h in range(8):
                v = plsc.load_gather(rows_vs[j], [rvs[h], cv])
                tbs[j][pl.ds(o + 16 * h, 16)] = v
            return c

        lax.fori_loop(0, _D, tbody, 0)

    def start_stores(r, j):
        rr = base + r
        s_pos = (rr // 1024) * 8 + rr % 8
        bt = (rr // 8) % 128
        o = (s_pos * 4 * 128 + bt) * 1024
        for dt in range(4):
            pltpu.async_copy(tbs[j].at[pl.ds(dt * 1024, 1024)],
                             out_hbm.at[pl.ds(o + dt * 131072, 1024)],
                             osems[j])

    def wait_stores(j):
        for dt in range(4):
            pltpu.make_async_copy(tbs[j].at[pl.ds(dt * 1024, 1024)],
                                  out_hbm.at[pl.ds(dt * 1024, 1024)],
                                  osems[j]).wait()

    for j in range(_NSLOT):
        start_idx(j, j)
    for j in range(_NSLOT):
        wait_idx(j)
        start_gather(j)

    for j in range(_NSLOT):
        wait_gather(j)
        start_idx(_NSLOT + j, j)
        transpose(j)
        start_stores(j, j)
        wait_idx(j)
        start_gather(j)

    def body(g, carry):
        for j in range(_NSLOT):
            wait_gather(j)
            start_idx((g + 1) * _NSLOT + j, j)
            wait_stores(j)
            transpose(j)
            start_stores(g * _NSLOT + j, j)
            wait_idx(j)
            start_gather(j)
        return carry

    lax.fori_loop(1, _NGRP - 1, body, 0)

    for j in range(_NSLOT):
        wait_gather(j)
        wait_stores(j)
        transpose(j)
        start_stores((_NGRP - 1) * _NSLOT + j, j)
    for j in range(_NSLOT):
        wait_stores(j)


def kernel(token_ids, embeddings):
    ids = jnp.transpose(
        token_ids.astype(jnp.int32).reshape(128, 128, 25, 8),
        (2, 0, 3, 1)).reshape(_NBLK, 128)
    out = _gather_kernel(ids, embeddings)
    t = out.reshape(_S, 4, 128, 8, 128)
    return jnp.transpose(t, (2, 4, 0, 1, 3)).reshape(_B, _S, _D)


# speedup vs baseline: 1.8881x; 1.6282x over previous
import functools

import jax
import jax.numpy as jnp
from jax import lax
from jax.experimental import pallas as pl
from jax.experimental.pallas import tpu as pltpu
from jax.experimental.pallas import tpu_sc as plsc

_B = 16384
_S = 200
_D = 32
_NTOK = _B * _S
_NBLK = _NTOK // 128
_NW = 32
_BPW = _NBLK // _NW
_NSLOT = 8
_NGRP = _BPW // _NSLOT

_mesh = plsc.VectorSubcoreMesh(core_axis_name="c", subcore_axis_name="s")


@functools.partial(
    pl.kernel,
    mesh=_mesh,
    compiler_params=pltpu.CompilerParams(use_tc_tiling_on_sc=False,
                                         needs_layout_passes=False),
    out_type=jax.ShapeDtypeStruct((_NTOK * _D,), jnp.float32),
    scratch_types=(
        [pltpu.VMEM((1, 128), jnp.int32) for _ in range(_NSLOT)]
        + [pltpu.VMEM((128, _D), jnp.float32) for _ in range(_NSLOT)]
        + [pltpu.VMEM((4096,), jnp.float32) for _ in range(_NSLOT)]
        + [pltpu.SemaphoreType.DMA for _ in range(3 * _NSLOT)]
    ),
)
def _gather_kernel(idx_hbm, table_hbm, out_hbm, *scratch):
    ixs = scratch[0:_NSLOT]
    rows_vs = scratch[_NSLOT:2 * _NSLOT]
    tbs = scratch[2 * _NSLOT:3 * _NSLOT]
    isems = scratch[3 * _NSLOT:4 * _NSLOT]
    gsems = scratch[4 * _NSLOT:5 * _NSLOT]
    osems = scratch[5 * _NSLOT:6 * _NSLOT]

    wid = lax.axis_index("s") * 2 + lax.axis_index("c")
    base = wid * _BPW

    iota = lax.iota(jnp.int32, 16)
    rvs = [iota + 16 * h for h in range(8)]

    def start_idx(r, j):
        pltpu.async_copy(idx_hbm.at[pl.ds(base + r, 1)], ixs[j], isems[j])

    def wait_idx(j):
        pltpu.make_async_copy(idx_hbm.at[pl.ds(0, 1)], ixs[j],
                              isems[j]).wait()

    def start_gather(j):
        pltpu.async_copy(table_hbm.at[ixs[j].at[0]], rows_vs[j], gsems[j])

    def wait_gather(j):
        pltpu.make_async_copy(table_hbm.at[ixs[j].at[0]], rows_vs[j],
                              gsems[j]).wait()

    def transpose(j):
        @plsc.parallel_loop(0, _D, unroll=4)
        def tbody(d):
            cv = jnp.full((16,), 1, jnp.int32) * d
            o = (d // 8) * 1024 + (d % 8) * 128
            for h in range(8):
                v = plsc.load_gather(rows_vs[j], [rvs[h], cv])
                tbs[j][pl.ds(o + 16 * h, 16)] = v

    def start_stores(r, j):
        rr = base + r
        s_pos = (rr // 1024) * 8 + rr % 8
        bt = (rr // 8) % 128
        o = (s_pos * 4 * 128 + bt) * 1024
        for dt in range(4):
            pltpu.async_copy(tbs[j].at[pl.ds(dt * 1024, 1024)],
                             out_hbm.at[pl.ds(o + dt * 131072, 1024)],
                             osems[j])

    def wait_stores(j):
        for dt in range(4):
            pltpu.make_async_copy(tbs[j].at[pl.ds(dt * 1024, 1024)],
                                  out_hbm.at[pl.ds(dt * 1024, 1024)],
                                  osems[j]).wait()

    for j in range(_NSLOT):
        start_idx(j, j)
    for j in range(_NSLOT):
        wait_idx(j)
        start_gather(j)

    for j in range(_NSLOT):
        wait_gather(j)
        start_idx(_NSLOT + j, j)
        transpose(j)
        start_stores(j, j)
        wait_idx(j)
        start_gather(j)

    def body(g, carry):
        for j in range(_NSLOT):
            wait_gather(j)
            start_idx((g + 1) * _NSLOT + j, j)
            wait_stores(j)
            transpose(j)
            start_stores(g * _NSLOT + j, j)
            wait_idx(j)
            start_gather(j)
        return carry

    lax.fori_loop(1, _NGRP - 1, body, 0)

    for j in range(_NSLOT):
        wait_gather(j)
        wait_stores(j)
        transpose(j)
        start_stores((_NGRP - 1) * _NSLOT + j, j)
    for j in range(_NSLOT):
        wait_stores(j)


def kernel(token_ids, embeddings):
    ids = jnp.transpose(
        token_ids.astype(jnp.int32).reshape(128, 128, 25, 8),
        (2, 0, 3, 1)).reshape(_NBLK, 128)
    out = _gather_kernel(ids, embeddings)
    t = out.reshape(_S, 4, 128, 8, 128)
    return jnp.transpose(t, (2, 4, 0, 1, 3)).reshape(_B, _S, _D)
